# Initial kernel scaffold; baseline (speedup 1.0000x reference)
import jax
import jax.numpy as jnp
from jax.experimental import pallas as pl


def kernel(x, Wg, bg, W1, b1, W2, b2, W3, b3):
    raise NotImplementedError("write your pallas kernel here")


# speedup vs baseline: 1.1695x; 1.1695x over previous
import jax
import jax.numpy as jnp
from jax.experimental import pallas as pl
from jax.experimental.pallas import tpu as pltpu


def _gate_kernel(x_ref, wg_ref, bg_ref, inp_ref):
    xb = x_ref[0]
    e = inp_ref.shape[1]
    d = xb.shape[1]
    k = inp_ref.shape[3] // d
    logits = jnp.dot(xb, wg_ref[...], preferred_element_type=jnp.float32)
    logits = logits + bg_ref[...]
    s = logits.shape[0]
    m = jnp.max(logits, axis=0, keepdims=True)
    denom = jnp.sum(jnp.exp(logits - m), axis=0, keepdims=True)
    iota = jax.lax.broadcasted_iota(jnp.int32, logits.shape, 0)
    cur = logits
    sel_cols = []
    for _ in range(k):
        vj = jnp.max(cur, axis=0, keepdims=True)
        aj = jnp.min(jnp.where(cur == vj, iota, s), axis=0, keepdims=True)
        selj = iota == aj
        pj = jnp.exp(vj - m) / denom
        sel_cols.append(jnp.where(selj, pj, 0.0)[:, :e])
        cur = jnp.where(selj, -jnp.inf, cur)
    w = jnp.concatenate(sel_cols, axis=1)
    rows = jax.lax.dot_general(
        w, xb, (((0,), (0,)), ((), ())), preferred_element_type=jnp.float32
    )
    for j in range(k):
        inp_ref[0, :, 0, j * d:(j + 1) * d] = rows[j * e:(j + 1) * e]


def _mlp_kernel(inp_ref, w1_ref, b1_ref, w2_ref, b2_ref, w3_ref, b3_ref,
                out_ref):
    b = inp_ref.shape[0]
    kd = inp_ref.shape[3]
    a = inp_ref[...].reshape(b, kd)
    h = jnp.dot(a, w1_ref[0], preferred_element_type=jnp.float32)
    h = jnp.maximum(h + b1_ref[0], 0.0)
    h = jnp.dot(h, w2_ref[0], preferred_element_type=jnp.float32)
    h = jnp.maximum(h + b2_ref[0], 0.0)
    h = jnp.dot(h, w3_ref[0], preferred_element_type=jnp.float32)
    h = h + b3_ref[0]
    out_ref[...] = h.reshape(out_ref.shape)


def kernel(x, Wg, bg, W1, b1, W2, b2, W3, b3):
    b, s, d = x.shape
    e = Wg.shape[1]
    kd = W1.shape[1]
    k = kd // d
    out_dim = W1.shape[2]

    ep = 128
    wg_p = jnp.zeros((d, ep), dtype=jnp.float32).at[:, :e].set(Wg)
    bg_p = jnp.zeros((1, ep), dtype=jnp.float32).at[0, :e].set(bg)

    inp = pl.pallas_call(
        _gate_kernel,
        grid=(b,),
        in_specs=[
            pl.BlockSpec((1, s, d), lambda i: (i, 0, 0)),
            pl.BlockSpec((d, ep), lambda i: (0, 0)),
            pl.BlockSpec((1, ep), lambda i: (0, 0)),
        ],
        out_specs=pl.BlockSpec((1, e, 1, kd), lambda i: (i, 0, 0, 0)),
        out_shape=jax.ShapeDtypeStruct((b, e, 1, kd), jnp.float32),
        compiler_params=pltpu.CompilerParams(
            dimension_semantics=("parallel",)),
    )(x, wg_p, bg_p)

    b1r = b1.reshape(e, 1, out_dim)
    b2r = b2.reshape(e, 1, out_dim)
    b3r = b3.reshape(e, 1, out_dim)

    out = pl.pallas_call(
        _mlp_kernel,
        grid=(e,),
        in_specs=[
            pl.BlockSpec((b, 1, 1, kd), lambda i: (0, i, 0, 0)),
            pl.BlockSpec((1, kd, out_dim), lambda i: (i, 0, 0)),
            pl.BlockSpec((1, 1, out_dim), lambda i: (i, 0, 0)),
            pl.BlockSpec((1, out_dim, out_dim), lambda i: (i, 0, 0)),
            pl.BlockSpec((1, 1, out_dim), lambda i: (i, 0, 0)),
            pl.BlockSpec((1, out_dim, out_dim), lambda i: (i, 0, 0)),
            pl.BlockSpec((1, 1, out_dim), lambda i: (i, 0, 0)),
        ],
        out_specs=pl.BlockSpec((b, 1, 1, out_dim), lambda i: (0, i, 0, 0)),
        out_shape=jax.ShapeDtypeStruct((b, e, 1, out_dim), jnp.float32),
        compiler_params=pltpu.CompilerParams(
            dimension_semantics=("arbitrary",)),
    )(inp, W1, b1r, W2, b2r, W3, b3r)

    return out.reshape(b, e, out_dim)
